# Initial kernel scaffold; baseline (speedup 1.0000x reference)
import functools

import jax
import jax.numpy as jnp
from jax import lax
from jax.experimental import pallas as pl
from jax.experimental.pallas import tpu as pltpu
from jax.experimental.pallas import tpu_sc as plsc

N = 10000
E = 320000
IN_CH = 128
HID = 16
OUT_CH = 128

NC = 2
NS = 16
NW = NC * NS
EPW = E // NW
EB = 125
KB = EPW // EB
RPS = N // NS

_mesh = plsc.VectorSubcoreMesh(core_axis_name="c", subcore_axis_name="s")


def _zero_rows(buf, nrows):
    z = jnp.zeros((16,), jnp.float32)

    def body(i, _):
        buf[i, :] = z
        return 0

    lax.fori_loop(0, nrows, body, 0)


@functools.partial(
    pl.kernel,
    out_type=jax.ShapeDtypeStruct((NC, N, HID), jnp.float32),
    mesh=_mesh,
    scratch_types=[
        pltpu.VMEM((KB, EB), jnp.int32),
        pltpu.VMEM((EB, HID), jnp.float32),
        pltpu.VMEM_SHARED((N, HID), jnp.float32),
    ],
)
def _deg_kernel(dst_hbm, out_hbm, dst_v, ones_v, acc):
    cid = lax.axis_index("c")
    sid = lax.axis_index("s")
    wid = cid * NS + sid

    _zero_rows(ones_v, EB)
    for t in range(RPS // EB):
        pltpu.sync_copy(ones_v, acc.at[pl.ds(sid * RPS + t * EB, EB)])

    one = jnp.ones((16,), jnp.float32)

    def fill(i, _):
        ones_v[i, :] = one
        return 0

    lax.fori_loop(0, EB, fill, 0)
    pltpu.sync_copy(dst_hbm.at[wid], dst_v)
    plsc.subcore_barrier()

    def edge_block(j, _):
        pltpu.sync_copy(ones_v, acc.at[dst_v.at[j]], add=True)
        return 0

    lax.fori_loop(0, KB, edge_block, 0)
    plsc.subcore_barrier()
    pltpu.sync_copy(acc.at[pl.ds(sid * RPS, RPS)],
                    out_hbm.at[cid, pl.ds(sid * RPS, RPS)])


@functools.partial(
    pl.kernel,
    out_type=jax.ShapeDtypeStruct((NC, N, HID), jnp.float32),
    mesh=_mesh,
    scratch_types=[
        pltpu.VMEM((KB, EB), jnp.int32),
        pltpu.VMEM((KB, EB), jnp.int32),
        pltpu.VMEM((EB, HID), jnp.float32),
        pltpu.VMEM_SHARED((N, HID), jnp.float32),
        pltpu.SemaphoreType.DMA,
    ],
)
def _agg_kernel(src_hbm, dst_hbm, feat_hbm, out_hbm,
                src_v, dst_v, rows_v, acc, sem):
    cid = lax.axis_index("c")
    sid = lax.axis_index("s")
    wid = cid * NS + sid

    _zero_rows(rows_v, EB)
    for t in range(RPS // EB):
        pltpu.sync_copy(rows_v, acc.at[pl.ds(sid * RPS + t * EB, EB)])
    pltpu.sync_copy(src_hbm.at[wid], src_v)
    pltpu.sync_copy(dst_hbm.at[wid], dst_v)
    plsc.subcore_barrier()

    def edge_block(j, _):
        pltpu.async_copy(feat_hbm.at[src_v.at[j]], rows_v, sem).wait()
        pltpu.sync_copy(rows_v, acc.at[dst_v.at[j]], add=True)
        return 0

    lax.fori_loop(0, KB, edge_block, 0)
    plsc.subcore_barrier()
    pltpu.sync_copy(acc.at[pl.ds(sid * RPS, RPS)],
                    out_hbm.at[cid, pl.ds(sid * RPS, RPS)])


_R = 2000


def _tc1_body(d0_ref, d1_ref, x_ref, w1_ref, g1_ref, dinv_ref):
    deg = d0_ref[...] + d1_ref[...] + 1.0
    dinv = lax.rsqrt(deg)
    h1 = jnp.dot(x_ref[...], w1_ref[...], preferred_element_type=jnp.float32)
    dinv_ref[...] = dinv
    g1_ref[...] = dinv * h1


def _tc1(d0, d1, x, W1):
    return pl.pallas_call(
        _tc1_body,
        grid=(N // _R,),
        in_specs=[
            pl.BlockSpec((_R, HID), lambda i: (i, 0)),
            pl.BlockSpec((_R, HID), lambda i: (i, 0)),
            pl.BlockSpec((_R, IN_CH), lambda i: (i, 0)),
            pl.BlockSpec((IN_CH, HID), lambda i: (0, 0)),
        ],
        out_specs=[
            pl.BlockSpec((_R, HID), lambda i: (i, 0)),
            pl.BlockSpec((_R, HID), lambda i: (i, 0)),
        ],
        out_shape=[
            jax.ShapeDtypeStruct((N, HID), jnp.float32),
            jax.ShapeDtypeStruct((N, HID), jnp.float32),
        ],
    )(d0, d1, x, W1)


def _tc2_body(s0_ref, s1_ref, g1_ref, dinv_ref, b1_ref, g2_ref):
    s = s0_ref[...] + s1_ref[...] + g1_ref[...]
    h = jnp.maximum(dinv_ref[...] * s + b1_ref[...], 0.0)
    g2_ref[...] = dinv_ref[...] * h


def _tc2(s0, s1, g1, dinv, b1):
    return pl.pallas_call(
        _tc2_body,
        grid=(N // _R,),
        in_specs=[
            pl.BlockSpec((_R, HID), lambda i: (i, 0)),
            pl.BlockSpec((_R, HID), lambda i: (i, 0)),
            pl.BlockSpec((_R, HID), lambda i: (i, 0)),
            pl.BlockSpec((_R, HID), lambda i: (i, 0)),
            pl.BlockSpec((1, HID), lambda i: (0, 0)),
        ],
        out_specs=pl.BlockSpec((_R, HID), lambda i: (i, 0)),
        out_shape=jax.ShapeDtypeStruct((N, HID), jnp.float32),
    )(s0, s1, g1, dinv, b1)


def _tc3_body(s0_ref, s1_ref, g2_ref, dinv_ref, w2_ref, b2_ref, out_ref):
    agg = dinv_ref[...] * (s0_ref[...] + s1_ref[...] + g2_ref[...])
    o = jnp.dot(agg, w2_ref[...], preferred_element_type=jnp.float32)
    o = o + b2_ref[...]
    m = jnp.max(o, axis=1, keepdims=True)
    lse = m + jnp.log(jnp.sum(jnp.exp(o - m), axis=1, keepdims=True))
    out_ref[...] = o - lse


def _tc3(s0, s1, g2, dinv, W2, b2):
    return pl.pallas_call(
        _tc3_body,
        grid=(N // _R,),
        in_specs=[
            pl.BlockSpec((_R, HID), lambda i: (i, 0)),
            pl.BlockSpec((_R, HID), lambda i: (i, 0)),
            pl.BlockSpec((_R, HID), lambda i: (i, 0)),
            pl.BlockSpec((_R, HID), lambda i: (i, 0)),
            pl.BlockSpec((HID, OUT_CH), lambda i: (0, 0)),
            pl.BlockSpec((1, OUT_CH), lambda i: (0, 0)),
        ],
        out_specs=pl.BlockSpec((_R, OUT_CH), lambda i: (i, 0)),
        out_shape=jax.ShapeDtypeStruct((N, OUT_CH), jnp.float32),
    )(s0, s1, g2, dinv, W2, b2)


def kernel(x, edge_index, W1, b1, W2, b2):
    src = edge_index[0].astype(jnp.int32).reshape(NW, KB, EB)
    dst = edge_index[1].astype(jnp.int32).reshape(NW, KB, EB)
    b1r = b1.reshape(1, HID)
    b2r = b2.reshape(1, OUT_CH)

    degp = _deg_kernel(dst)
    g1, dinv = _tc1(degp[0], degp[1], x, W1)
    s1p = _agg_kernel(src, dst, g1)
    g2 = _tc2(s1p[0], s1p[1], g1, dinv, b1r)
    s2p = _agg_kernel(src, dst, g2)
    return _tc3(s2p[0], s2p[1], g2, dinv, W2, b2)


# speedup vs baseline: 26.8328x; 26.8328x over previous
import functools

import jax
import jax.numpy as jnp
from jax import lax
from jax.experimental import pallas as pl
from jax.experimental.pallas import tpu as pltpu
from jax.experimental.pallas import tpu_sc as plsc

N = 10000
E = 320000
IN_CH = 128
HID = 16
OUT_CH = 128

NC = 2
NS = 16
NW = NC * NS
EB = 128
KB = 80
E_PAD = NW * KB * EB
N_PAD = 10240
RPS = N_PAD // NS
ZCH = RPS // EB

_mesh = plsc.VectorSubcoreMesh(core_axis_name="c", subcore_axis_name="s")


def _fill_rows(buf, nrows, vec):
    def body(i, _):
        buf[i, :] = vec
        return 0

    lax.fori_loop(0, nrows, body, 0)


@functools.partial(
    pl.kernel,
    out_type=jax.ShapeDtypeStruct((NC, N_PAD, HID), jnp.float32),
    mesh=_mesh,
    scratch_types=[
        pltpu.VMEM((KB, EB), jnp.int32),
        pltpu.VMEM((EB, HID), jnp.float32),
        pltpu.VMEM_SHARED((N_PAD, HID), jnp.float32),
    ],
    compiler_params=pltpu.CompilerParams(use_tc_tiling_on_sc=False),
)
def _deg_kernel(dst_hbm, out_hbm, dst_v, ones_v, acc):
    cid = lax.axis_index("c")
    sid = lax.axis_index("s")
    wid = cid * NS + sid

    _fill_rows(ones_v, EB, jnp.zeros((16,), jnp.float32))
    for t in range(ZCH):
        pltpu.sync_copy(ones_v, acc.at[pl.ds(sid * RPS + t * EB, EB)])
    _fill_rows(ones_v, EB, jnp.ones((16,), jnp.float32))
    pltpu.sync_copy(dst_hbm.at[wid], dst_v)
    plsc.subcore_barrier()

    def edge_block(j, _):
        pltpu.sync_copy(ones_v, acc.at[dst_v.at[j]], add=True)
        return 0

    lax.fori_loop(0, KB, edge_block, 0)
    plsc.subcore_barrier()
    pltpu.sync_copy(acc.at[pl.ds(sid * RPS, RPS)],
                    out_hbm.at[cid, pl.ds(sid * RPS, RPS)])


@functools.partial(
    pl.kernel,
    out_type=jax.ShapeDtypeStruct((NC, N_PAD, HID), jnp.float32),
    mesh=_mesh,
    scratch_types=[
        pltpu.VMEM((KB, EB), jnp.int32),
        pltpu.VMEM((KB, EB), jnp.int32),
        pltpu.VMEM((EB, HID), jnp.float32),
        pltpu.VMEM_SHARED((N_PAD, HID), jnp.float32),
        pltpu.SemaphoreType.DMA,
    ],
    compiler_params=pltpu.CompilerParams(use_tc_tiling_on_sc=False),
)
def _agg_kernel(src_hbm, dst_hbm, feat_hbm, out_hbm,
                src_v, dst_v, rows_v, acc, sem):
    cid = lax.axis_index("c")
    sid = lax.axis_index("s")
    wid = cid * NS + sid

    _fill_rows(rows_v, EB, jnp.zeros((16,), jnp.float32))
    for t in range(ZCH):
        pltpu.sync_copy(rows_v, acc.at[pl.ds(sid * RPS + t * EB, EB)])
    pltpu.sync_copy(src_hbm.at[wid], src_v)
    pltpu.sync_copy(dst_hbm.at[wid], dst_v)
    plsc.subcore_barrier()

    def edge_block(j, _):
        pltpu.async_copy(feat_hbm.at[src_v.at[j]], rows_v, sem).wait()
        pltpu.sync_copy(rows_v, acc.at[dst_v.at[j]], add=True)
        return 0

    lax.fori_loop(0, KB, edge_block, 0)
    plsc.subcore_barrier()
    pltpu.sync_copy(acc.at[pl.ds(sid * RPS, RPS)],
                    out_hbm.at[cid, pl.ds(sid * RPS, RPS)])


_R = 2048
_R3 = 2000


def _tc1_body(d0_ref, d1_ref, x_ref, w1_ref, g1_ref, dinv_ref):
    deg = d0_ref[...] + d1_ref[...] + 1.0
    dinv = lax.rsqrt(deg)
    h1 = jnp.dot(x_ref[...], w1_ref[...], preferred_element_type=jnp.float32)
    dinv_ref[...] = dinv
    g1_ref[...] = dinv * h1


def _tc1(d0, d1, x, W1):
    return pl.pallas_call(
        _tc1_body,
        grid=(N_PAD // _R,),
        in_specs=[
            pl.BlockSpec((_R, HID), lambda i: (i, 0)),
            pl.BlockSpec((_R, HID), lambda i: (i, 0)),
            pl.BlockSpec((_R, IN_CH), lambda i: (i, 0)),
            pl.BlockSpec((IN_CH, HID), lambda i: (0, 0)),
        ],
        out_specs=[
            pl.BlockSpec((_R, HID), lambda i: (i, 0)),
            pl.BlockSpec((_R, HID), lambda i: (i, 0)),
        ],
        out_shape=[
            jax.ShapeDtypeStruct((N_PAD, HID), jnp.float32),
            jax.ShapeDtypeStruct((N_PAD, HID), jnp.float32),
        ],
    )(d0, d1, x, W1)


def _tc2_body(s0_ref, s1_ref, g1_ref, dinv_ref, b1_ref, g2_ref):
    s = s0_ref[...] + s1_ref[...] + g1_ref[...]
    h = jnp.maximum(dinv_ref[...] * s + b1_ref[...], 0.0)
    g2_ref[...] = dinv_ref[...] * h


def _tc2(s0, s1, g1, dinv, b1):
    return pl.pallas_call(
        _tc2_body,
        grid=(N_PAD // _R,),
        in_specs=[
            pl.BlockSpec((_R, HID), lambda i: (i, 0)),
            pl.BlockSpec((_R, HID), lambda i: (i, 0)),
            pl.BlockSpec((_R, HID), lambda i: (i, 0)),
            pl.BlockSpec((_R, HID), lambda i: (i, 0)),
            pl.BlockSpec((1, HID), lambda i: (0, 0)),
        ],
        out_specs=pl.BlockSpec((_R, HID), lambda i: (i, 0)),
        out_shape=jax.ShapeDtypeStruct((N_PAD, HID), jnp.float32),
    )(s0, s1, g1, dinv, b1)


def _tc3_body(s0_ref, s1_ref, g2_ref, dinv_ref, w2_ref, b2_ref, out_ref):
    agg = dinv_ref[...] * (s0_ref[...] + s1_ref[...] + g2_ref[...])
    o = jnp.dot(agg, w2_ref[...], preferred_element_type=jnp.float32)
    o = o + b2_ref[...]
    m = jnp.max(o, axis=1, keepdims=True)
    lse = m + jnp.log(jnp.sum(jnp.exp(o - m), axis=1, keepdims=True))
    out_ref[...] = o - lse


def _tc3(s0, s1, g2, dinv, W2, b2):
    return pl.pallas_call(
        _tc3_body,
        grid=(N // _R3,),
        in_specs=[
            pl.BlockSpec((_R3, HID), lambda i: (i, 0)),
            pl.BlockSpec((_R3, HID), lambda i: (i, 0)),
            pl.BlockSpec((_R3, HID), lambda i: (i, 0)),
            pl.BlockSpec((_R3, HID), lambda i: (i, 0)),
            pl.BlockSpec((HID, OUT_CH), lambda i: (0, 0)),
            pl.BlockSpec((1, OUT_CH), lambda i: (0, 0)),
        ],
        out_specs=pl.BlockSpec((_R3, OUT_CH), lambda i: (i, 0)),
        out_shape=jax.ShapeDtypeStruct((N, OUT_CH), jnp.float32),
    )(s0, s1, g2, dinv, W2, b2)


def kernel(x, edge_index, W1, b1, W2, b2):
    pad = jnp.full((E_PAD - E,), N, jnp.int32)
    src = jnp.concatenate([edge_index[0].astype(jnp.int32), pad])
    dst = jnp.concatenate([edge_index[1].astype(jnp.int32), pad])
    src = src.reshape(NW, KB, EB)
    dst = dst.reshape(NW, KB, EB)
    xp = jnp.zeros((N_PAD, IN_CH), x.dtype).at[:N].set(x)
    b1r = b1.reshape(1, HID)
    b2r = b2.reshape(1, OUT_CH)

    degp = _deg_kernel(dst)
    g1, dinv = _tc1(degp[0], degp[1], xp, W1)
    s1p = _agg_kernel(src, dst, g1)
    g2 = _tc2(s1p[0], s1p[1], g1, dinv, b1r)
    s2p = _agg_kernel(src, dst, g2)
    return _tc3(s2p[0], s2p[1], g2, dinv, W2, b2r)


# speedup vs baseline: 34.4972x; 1.2856x over previous
import functools

import jax
import jax.numpy as jnp
from jax import lax
from jax.experimental import pallas as pl
from jax.experimental.pallas import tpu as pltpu
from jax.experimental.pallas import tpu_sc as plsc

N = 10000
E = 320000
IN_CH = 128
HID = 16
OUT_CH = 128

NC = 2
NS = 16
NW = NC * NS
EB = 128
KB = 80
E_PAD = NW * KB * EB
N_PAD = 10240
RPS = N_PAD // NS
ZCH = RPS // EB
NBUF = 4
NCH = KB // NBUF

_mesh = plsc.VectorSubcoreMesh(core_axis_name="c", subcore_axis_name="s")


def _fill_rows(buf, nrows, vec):
    def body(i, _):
        buf[i, :] = vec
        return 0

    lax.fori_loop(0, nrows, body, 0)


@functools.partial(
    pl.kernel,
    out_type=jax.ShapeDtypeStruct((NC, N_PAD, HID), jnp.float32),
    mesh=_mesh,
    scratch_types=[
        pltpu.VMEM((KB, EB), jnp.int32),
        pltpu.VMEM((EB, HID), jnp.float32),
        pltpu.VMEM_SHARED((N_PAD, HID), jnp.float32),
        pltpu.SemaphoreType.DMA,
    ],
    compiler_params=pltpu.CompilerParams(use_tc_tiling_on_sc=False),
)
def _deg_kernel(dst_hbm, out_hbm, dst_v, ones_v, acc, sem):
    cid = lax.axis_index("c")
    sid = lax.axis_index("s")
    wid = cid * NS + sid

    _fill_rows(ones_v, EB, jnp.zeros((16,), jnp.float32))
    for t in range(ZCH):
        pltpu.sync_copy(ones_v, acc.at[pl.ds(sid * RPS + t * EB, EB)])
    _fill_rows(ones_v, EB, jnp.ones((16,), jnp.float32))
    pltpu.sync_copy(dst_hbm.at[wid], dst_v)
    plsc.subcore_barrier()

    def edge_block(j, _):
        @pl.when(j >= 16)
        def _():
            pltpu.make_async_copy(ones_v, acc.at[dst_v.at[j - 16]], sem).wait()

        pltpu.async_copy(ones_v, acc.at[dst_v.at[j]], sem, add=True)
        return 0

    lax.fori_loop(0, KB, edge_block, 0)
    for t in range(16):
        pltpu.make_async_copy(ones_v, acc.at[dst_v.at[KB - 16 + t]], sem).wait()
    plsc.subcore_barrier()
    pltpu.sync_copy(acc.at[pl.ds(sid * RPS, RPS)],
                    out_hbm.at[cid, pl.ds(sid * RPS, RPS)])


@functools.partial(
    pl.kernel,
    out_type=jax.ShapeDtypeStruct((NC, N_PAD, HID), jnp.float32),
    mesh=_mesh,
    scratch_types=[
        pltpu.VMEM((KB, EB), jnp.int32),
        pltpu.VMEM((KB, EB), jnp.int32),
        pltpu.VMEM((2, NBUF, EB, HID), jnp.float32),
        pltpu.VMEM((EB, HID), jnp.float32),
        pltpu.VMEM_SHARED((N_PAD, HID), jnp.float32),
        pltpu.SemaphoreType.DMA,
        pltpu.SemaphoreType.DMA,
        pltpu.SemaphoreType.DMA,
        pltpu.SemaphoreType.DMA,
    ],
    compiler_params=pltpu.CompilerParams(use_tc_tiling_on_sc=False),
)
def _agg_kernel(src_hbm, dst_hbm, feat_hbm, out_hbm,
                src_v, dst_v, rows_v, zero_v, acc, gs0, gs1, ss0, ss1):
    cid = lax.axis_index("c")
    sid = lax.axis_index("s")
    wid = cid * NS + sid

    _fill_rows(zero_v, EB, jnp.zeros((16,), jnp.float32))
    for t in range(ZCH):
        pltpu.sync_copy(zero_v, acc.at[pl.ds(sid * RPS + t * EB, EB)])
    pltpu.sync_copy(src_hbm.at[wid], src_v)
    pltpu.sync_copy(dst_hbm.at[wid], dst_v)
    plsc.subcore_barrier()

    def gather(c, p, sem):
        for b in range(NBUF):
            pltpu.async_copy(feat_hbm.at[src_v.at[c * NBUF + b]],
                             rows_v.at[p, b], sem)

    def wait_gather(c, p, sem):
        for b in range(NBUF):
            pltpu.make_async_copy(feat_hbm.at[src_v.at[c * NBUF + b]],
                                  rows_v.at[p, b], sem).wait()

    def scatter(c, p, sem):
        for b in range(NBUF):
            pltpu.async_copy(rows_v.at[p, b],
                             acc.at[dst_v.at[c * NBUF + b]], sem, add=True)

    def wait_scatter(c, p, sem):
        for b in range(NBUF):
            pltpu.make_async_copy(rows_v.at[p, b],
                                  acc.at[dst_v.at[c * NBUF + b]], sem).wait()

    gather(0, 0, gs0)

    def body(i, _):
        c0 = 2 * i
        c1 = c0 + 1

        @pl.when(i > 0)
        def _():
            wait_scatter(c0 - 1, 1, ss1)

        gather(c1, 1, gs1)
        wait_gather(c0, 0, gs0)
        scatter(c0, 0, ss0)
        wait_gather(c1, 1, gs1)
        scatter(c1, 1, ss1)
        wait_scatter(c0, 0, ss0)

        @pl.when(i < NCH // 2 - 1)
        def _():
            gather(c0 + 2, 0, gs0)

        return 0

    lax.fori_loop(0, NCH // 2, body, 0)
    wait_scatter(NCH - 1, 1, ss1)
    plsc.subcore_barrier()
    pltpu.sync_copy(acc.at[pl.ds(sid * RPS, RPS)],
                    out_hbm.at[cid, pl.ds(sid * RPS, RPS)])


_R = 2048
_R3 = 2000


def _tc1_body(d0_ref, d1_ref, x_ref, w1_ref, g1_ref, dinv_ref):
    deg = d0_ref[...] + d1_ref[...] + 1.0
    dinv = lax.rsqrt(deg)
    h1 = jnp.dot(x_ref[...], w1_ref[...], preferred_element_type=jnp.float32)
    dinv_ref[...] = dinv
    g1_ref[...] = dinv * h1


def _tc1(d0, d1, x, W1):
    return pl.pallas_call(
        _tc1_body,
        grid=(N_PAD // _R,),
        in_specs=[
            pl.BlockSpec((_R, HID), lambda i: (i, 0)),
            pl.BlockSpec((_R, HID), lambda i: (i, 0)),
            pl.BlockSpec((_R, IN_CH), lambda i: (i, 0)),
            pl.BlockSpec((IN_CH, HID), lambda i: (0, 0)),
        ],
        out_specs=[
            pl.BlockSpec((_R, HID), lambda i: (i, 0)),
            pl.BlockSpec((_R, HID), lambda i: (i, 0)),
        ],
        out_shape=[
            jax.ShapeDtypeStruct((N_PAD, HID), jnp.float32),
            jax.ShapeDtypeStruct((N_PAD, HID), jnp.float32),
        ],
    )(d0, d1, x, W1)


def _tc2_body(s0_ref, s1_ref, g1_ref, dinv_ref, b1_ref, g2_ref):
    s = s0_ref[...] + s1_ref[...] + g1_ref[...]
    h = jnp.maximum(dinv_ref[...] * s + b1_ref[...], 0.0)
    g2_ref[...] = dinv_ref[...] * h


def _tc2(s0, s1, g1, dinv, b1):
    return pl.pallas_call(
        _tc2_body,
        grid=(N_PAD // _R,),
        in_specs=[
            pl.BlockSpec((_R, HID), lambda i: (i, 0)),
            pl.BlockSpec((_R, HID), lambda i: (i, 0)),
            pl.BlockSpec((_R, HID), lambda i: (i, 0)),
            pl.BlockSpec((_R, HID), lambda i: (i, 0)),
            pl.BlockSpec((1, HID), lambda i: (0, 0)),
        ],
        out_specs=pl.BlockSpec((_R, HID), lambda i: (i, 0)),
        out_shape=jax.ShapeDtypeStruct((N_PAD, HID), jnp.float32),
    )(s0, s1, g1, dinv, b1)


def _tc3_body(s0_ref, s1_ref, g2_ref, dinv_ref, w2_ref, b2_ref, out_ref):
    agg = dinv_ref[...] * (s0_ref[...] + s1_ref[...] + g2_ref[...])
    o = jnp.dot(agg, w2_ref[...], preferred_element_type=jnp.float32)
    o = o + b2_ref[...]
    m = jnp.max(o, axis=1, keepdims=True)
    lse = m + jnp.log(jnp.sum(jnp.exp(o - m), axis=1, keepdims=True))
    out_ref[...] = o - lse


def _tc3(s0, s1, g2, dinv, W2, b2):
    return pl.pallas_call(
        _tc3_body,
        grid=(N // _R3,),
        in_specs=[
            pl.BlockSpec((_R3, HID), lambda i: (i, 0)),
            pl.BlockSpec((_R3, HID), lambda i: (i, 0)),
            pl.BlockSpec((_R3, HID), lambda i: (i, 0)),
            pl.BlockSpec((_R3, HID), lambda i: (i, 0)),
            pl.BlockSpec((HID, OUT_CH), lambda i: (0, 0)),
            pl.BlockSpec((1, OUT_CH), lambda i: (0, 0)),
        ],
        out_specs=pl.BlockSpec((_R3, OUT_CH), lambda i: (i, 0)),
        out_shape=jax.ShapeDtypeStruct((N, OUT_CH), jnp.float32),
    )(s0, s1, g2, dinv, W2, b2)


def kernel(x, edge_index, W1, b1, W2, b2):
    pad = jnp.full((E_PAD - E,), N, jnp.int32)
    src = jnp.concatenate([edge_index[0].astype(jnp.int32), pad])
    dst = jnp.concatenate([edge_index[1].astype(jnp.int32), pad])
    src = src.reshape(NW, KB, EB)
    dst = dst.reshape(NW, KB, EB)
    xp = jnp.zeros((N_PAD, IN_CH), x.dtype).at[:N].set(x)
    b1r = b1.reshape(1, HID)
    b2r = b2.reshape(1, OUT_CH)

    degp = _deg_kernel(dst)
    g1, dinv = _tc1(degp[0], degp[1], xp, W1)
    s1p = _agg_kernel(src, dst, g1)
    g2 = _tc2(s1p[0], s1p[1], g1, dinv, b1r)
    s2p = _agg_kernel(src, dst, g2)
    return _tc3(s2p[0], s2p[1], g2, dinv, W2, b2r)


# speedup vs baseline: 50.2707x; 1.4572x over previous
import functools

import jax
import jax.numpy as jnp
from jax import lax
from jax.experimental import pallas as pl
from jax.experimental.pallas import tpu as pltpu
from jax.experimental.pallas import tpu_sc as plsc

N = 10000
E = 320000
IN_CH = 128
HID = 16
OUT_CH = 128

NC = 2
NS = 16
NW = NC * NS
EB = 128
KB = 80
E_PAD = NW * KB * EB
N_PAD = 10240
RPS = N_PAD // NS
ZCH = RPS // EB
NBUF = 4
NCH = KB // NBUF

_mesh = plsc.VectorSubcoreMesh(core_axis_name="c", subcore_axis_name="s")


def _fill_rows(buf, nrows, vec):
    def body(i, _):
        buf[i, :] = vec
        return 0

    lax.fori_loop(0, nrows, body, 0)


@functools.partial(
    pl.kernel,
    out_type=jax.ShapeDtypeStruct((NC, N_PAD, HID), jnp.float32),
    mesh=_mesh,
    scratch_types=[
        pltpu.VMEM((KB, EB), jnp.int32),
        pltpu.VMEM((EB, HID), jnp.float32),
        pltpu.VMEM_SHARED((N_PAD, HID), jnp.float32),
        pltpu.SemaphoreType.DMA,
    ],
    compiler_params=pltpu.CompilerParams(use_tc_tiling_on_sc=False),
)
def _deg_kernel(dst_hbm, out_hbm, dst_v, ones_v, acc, sem):
    cid = lax.axis_index("c")
    sid = lax.axis_index("s")
    wid = cid * NS + sid

    _fill_rows(ones_v, EB, jnp.zeros((16,), jnp.float32))
    for t in range(ZCH):
        pltpu.sync_copy(ones_v, acc.at[pl.ds(sid * RPS + t * EB, EB)])
    _fill_rows(ones_v, EB, jnp.ones((16,), jnp.float32))
    pltpu.sync_copy(dst_hbm.at[wid], dst_v)
    plsc.subcore_barrier()

    def edge_block(j, _):
        @pl.when(j >= 16)
        def _():
            pltpu.make_async_copy(ones_v, acc.at[dst_v.at[j - 16]], sem).wait()

        pltpu.async_copy(ones_v, acc.at[dst_v.at[j]], sem, add=True)
        return 0

    lax.fori_loop(0, KB, edge_block, 0)
    for t in range(16):
        pltpu.make_async_copy(ones_v, acc.at[dst_v.at[KB - 16 + t]], sem).wait()
    plsc.subcore_barrier()
    pltpu.sync_copy(acc.at[pl.ds(sid * RPS, RPS)],
                    out_hbm.at[cid, pl.ds(sid * RPS, RPS)])


@functools.partial(
    pl.kernel,
    out_type=jax.ShapeDtypeStruct((NC, N_PAD, HID), jnp.float32),
    mesh=_mesh,
    scratch_types=[
        pltpu.VMEM((KB, EB), jnp.int32),
        pltpu.VMEM((KB, EB), jnp.int32),
        pltpu.VMEM((2, NBUF, EB, HID), jnp.float32),
        pltpu.VMEM((EB, HID), jnp.float32),
        pltpu.VMEM_SHARED((N_PAD, HID), jnp.float32),
        pltpu.VMEM_SHARED((N_PAD, HID), jnp.float32),
        pltpu.SemaphoreType.DMA,
        pltpu.SemaphoreType.DMA,
        pltpu.SemaphoreType.DMA,
        pltpu.SemaphoreType.DMA,
    ],
    compiler_params=pltpu.CompilerParams(use_tc_tiling_on_sc=False),
)
def _agg_kernel(src_hbm, dst_hbm, feat_hbm, out_hbm,
                src_v, dst_v, rows_v, zero_v, acc, g_sp, gs0, gs1, ss0, ss1):
    cid = lax.axis_index("c")
    sid = lax.axis_index("s")
    wid = cid * NS + sid

    pltpu.sync_copy(feat_hbm.at[pl.ds(sid * RPS, RPS)],
                    g_sp.at[pl.ds(sid * RPS, RPS)])
    _fill_rows(zero_v, EB, jnp.zeros((16,), jnp.float32))
    for t in range(ZCH):
        pltpu.sync_copy(zero_v, acc.at[pl.ds(sid * RPS + t * EB, EB)])
    pltpu.sync_copy(src_hbm.at[wid], src_v)
    pltpu.sync_copy(dst_hbm.at[wid], dst_v)
    plsc.subcore_barrier()

    def gather(c, p, sem):
        for b in range(NBUF):
            pltpu.async_copy(g_sp.at[src_v.at[c * NBUF + b]],
                             rows_v.at[p, b], sem)

    def wait_gather(c, p, sem):
        for b in range(NBUF):
            pltpu.make_async_copy(g_sp.at[src_v.at[c * NBUF + b]],
                                  rows_v.at[p, b], sem).wait()

    def scatter(c, p, sem):
        for b in range(NBUF):
            pltpu.async_copy(rows_v.at[p, b],
                             acc.at[dst_v.at[c * NBUF + b]], sem, add=True)

    def wait_scatter(c, p, sem):
        for b in range(NBUF):
            pltpu.make_async_copy(rows_v.at[p, b],
                                  acc.at[dst_v.at[c * NBUF + b]], sem).wait()

    gather(0, 0, gs0)

    def body(i, _):
        c0 = 2 * i
        c1 = c0 + 1

        @pl.when(i > 0)
        def _():
            wait_scatter(c0 - 1, 1, ss1)

        gather(c1, 1, gs1)
        wait_gather(c0, 0, gs0)
        scatter(c0, 0, ss0)
        wait_gather(c1, 1, gs1)
        scatter(c1, 1, ss1)
        wait_scatter(c0, 0, ss0)

        @pl.when(i < NCH // 2 - 1)
        def _():
            gather(c0 + 2, 0, gs0)

        return 0

    lax.fori_loop(0, NCH // 2, body, 0)
    wait_scatter(NCH - 1, 1, ss1)
    plsc.subcore_barrier()
    pltpu.sync_copy(acc.at[pl.ds(sid * RPS, RPS)],
                    out_hbm.at[cid, pl.ds(sid * RPS, RPS)])


_R = 2048
_R3 = 2000


def _tc1_body(d0_ref, d1_ref, x_ref, w1_ref, g1_ref, dinv_ref):
    deg = d0_ref[...] + d1_ref[...] + 1.0
    dinv = lax.rsqrt(deg)
    h1 = jnp.dot(x_ref[...], w1_ref[...], preferred_element_type=jnp.float32)
    dinv_ref[...] = dinv
    g1_ref[...] = dinv * h1


def _tc1(d0, d1, x, W1):
    return pl.pallas_call(
        _tc1_body,
        grid=(N_PAD // _R,),
        in_specs=[
            pl.BlockSpec((_R, HID), lambda i: (i, 0)),
            pl.BlockSpec((_R, HID), lambda i: (i, 0)),
            pl.BlockSpec((_R, IN_CH), lambda i: (i, 0)),
            pl.BlockSpec((IN_CH, HID), lambda i: (0, 0)),
        ],
        out_specs=[
            pl.BlockSpec((_R, HID), lambda i: (i, 0)),
            pl.BlockSpec((_R, HID), lambda i: (i, 0)),
        ],
        out_shape=[
            jax.ShapeDtypeStruct((N_PAD, HID), jnp.float32),
            jax.ShapeDtypeStruct((N_PAD, HID), jnp.float32),
        ],
    )(d0, d1, x, W1)


def _tc2_body(s0_ref, s1_ref, g1_ref, dinv_ref, b1_ref, g2_ref):
    s = s0_ref[...] + s1_ref[...] + g1_ref[...]
    h = jnp.maximum(dinv_ref[...] * s + b1_ref[...], 0.0)
    g2_ref[...] = dinv_ref[...] * h


def _tc2(s0, s1, g1, dinv, b1):
    return pl.pallas_call(
        _tc2_body,
        grid=(N_PAD // _R,),
        in_specs=[
            pl.BlockSpec((_R, HID), lambda i: (i, 0)),
            pl.BlockSpec((_R, HID), lambda i: (i, 0)),
            pl.BlockSpec((_R, HID), lambda i: (i, 0)),
            pl.BlockSpec((_R, HID), lambda i: (i, 0)),
            pl.BlockSpec((1, HID), lambda i: (0, 0)),
        ],
        out_specs=pl.BlockSpec((_R, HID), lambda i: (i, 0)),
        out_shape=jax.ShapeDtypeStruct((N_PAD, HID), jnp.float32),
    )(s0, s1, g1, dinv, b1)


def _tc3_body(s0_ref, s1_ref, g2_ref, dinv_ref, w2_ref, b2_ref, out_ref):
    agg = dinv_ref[...] * (s0_ref[...] + s1_ref[...] + g2_ref[...])
    o = jnp.dot(agg, w2_ref[...], preferred_element_type=jnp.float32)
    o = o + b2_ref[...]
    m = jnp.max(o, axis=1, keepdims=True)
    lse = m + jnp.log(jnp.sum(jnp.exp(o - m), axis=1, keepdims=True))
    out_ref[...] = o - lse


def _tc3(s0, s1, g2, dinv, W2, b2):
    return pl.pallas_call(
        _tc3_body,
        grid=(N // _R3,),
        in_specs=[
            pl.BlockSpec((_R3, HID), lambda i: (i, 0)),
            pl.BlockSpec((_R3, HID), lambda i: (i, 0)),
            pl.BlockSpec((_R3, HID), lambda i: (i, 0)),
            pl.BlockSpec((_R3, HID), lambda i: (i, 0)),
            pl.BlockSpec((HID, OUT_CH), lambda i: (0, 0)),
            pl.BlockSpec((1, OUT_CH), lambda i: (0, 0)),
        ],
        out_specs=pl.BlockSpec((_R3, OUT_CH), lambda i: (i, 0)),
        out_shape=jax.ShapeDtypeStruct((N, OUT_CH), jnp.float32),
    )(s0, s1, g2, dinv, W2, b2)


def kernel(x, edge_index, W1, b1, W2, b2):
    pad = jnp.full((E_PAD - E,), N, jnp.int32)
    src = jnp.concatenate([edge_index[0].astype(jnp.int32), pad])
    dst = jnp.concatenate([edge_index[1].astype(jnp.int32), pad])
    src = src.reshape(NW, KB, EB)
    dst = dst.reshape(NW, KB, EB)
    xp = jnp.zeros((N_PAD, IN_CH), x.dtype).at[:N].set(x)
    b1r = b1.reshape(1, HID)
    b2r = b2.reshape(1, OUT_CH)

    degp = _deg_kernel(dst)
    g1, dinv = _tc1(degp[0], degp[1], xp, W1)
    s1p = _agg_kernel(src, dst, g1)
    g2 = _tc2(s1p[0], s1p[1], g1, dinv, b1r)
    s2p = _agg_kernel(src, dst, g2)
    return _tc3(s2p[0], s2p[1], g2, dinv, W2, b2r)
